# Initial kernel scaffold; baseline (speedup 1.0000x reference)
import jax
import jax.numpy as jnp
from jax.experimental import pallas as pl


def kernel(x, edge, sensor_indx, params):
    raise NotImplementedError("write your pallas kernel here")


# speedup vs baseline: 1.0274x; 1.0274x over previous
import jax
import jax.numpy as jnp
from jax.experimental import pallas as pl

NPAD = 64
NV = 51
H = 4


def _leaky(z):
    return jnp.where(z >= 0, z, 0.2 * z)


def _elu(z):
    return jnp.where(z > 0, z, jnp.exp(jnp.where(z > 0, 0.0, z)) - 1.0)


def _b16(v):
    return v.astype(jnp.bfloat16)


def _rf(v):
    return v.astype(jnp.bfloat16).astype(jnp.float32)


def _dot(a, b):
    return jnp.dot(_b16(a), _b16(b), preferred_element_type=jnp.float32)


def _gat(xin, mT, wl, wr, att, b, d):
    xl = _dot(xin, wl)
    xr = _dot(xin, wr)
    acc = jnp.zeros((NPAD, d), jnp.float32)
    for h in range(H):
        xlh = xl[:, h * d:(h + 1) * d]
        xrh = xr[:, h * d:(h + 1) * d]
        z = xlh[None, :, :] + xrh[:, None, :]
        e = _leaky(z)
        atth = _rf(att[h:h + 1, :]).reshape(1, 1, d)
        s = jnp.sum(_rf(e) * atth, axis=-1)
        s = jnp.where(mT, s, -1e9)
        s = s - jnp.max(s, axis=1, keepdims=True)
        ex = jnp.exp(s)
        a = ex / jnp.sum(ex, axis=1, keepdims=True)
        a = jnp.where(mT, a, 0.0)
        acc = acc + _dot(a, xlh)
    return acc * (1.0 / H) + b


def _body(names, *refs):
    n_in = len(names)
    V = {nm: refs[i][...] for i, nm in enumerate(names)}
    recon_ref, fc_ref, nr_ref = refs[n_in:]

    mTA = V['eTA'] != 0.0
    mTB = V['eTB'] != 0.0

    def make_xc(xt):
        xp = jnp.sum(_rf(xt)[:, :, None] * _rf(V['proj']), axis=1)
        return jnp.concatenate([xp, V['emb']], axis=-1)

    xcA = make_xc(V['xA'])
    xcB = make_xc(V['xB'])

    def conv_stack(xc, mT):
        o = _elu(_gat(xc, mT, V['c1Wl'], V['c1Wr'], V['c1att'], V['c1b'], 128))
        E = _elu(_gat(o, mT, V['c2Wl'], V['c2Wr'], V['c2att'], V['c2b'], 64))
        E = _elu(jnp.concatenate(
            [_gat(E, mT, V['c3Wl'], V['c3Wr'], V['c3att'], V['c3b'], 64), E],
            axis=-1))
        E = _elu(_gat(E, mT, V['c4Wl'], V['c4Wr'], V['c4att'], V['c4b'], 64))
        return E

    EA = conv_stack(xcA, mTA)
    EB = conv_stack(xcB, mTB)

    r = _elu(_gat(EB, mTB, V['r11Wl'], V['r11Wr'], V['r11att'], V['r11b'], 64))
    r = _elu(jnp.concatenate(
        [_gat(r, mTB, V['r12Wl'], V['r12Wr'], V['r12att'], V['r12b'], 64), r],
        axis=-1))
    r = _elu(_gat(r, mTB, V['r2Wl'], V['r2Wr'], V['r2att'], V['r2b'], 128))
    r = jnp.tanh(_dot(r, V['r3W']) + V['r3b'])
    recon_ref[...] = (jnp.sum(_rf(r) * _rf(V['r4Wr']), axis=1, keepdims=True)
                      + V['r4b'][0, 0])

    f = _elu(_gat(EA, mTA, V['f11Wl'], V['f11Wr'], V['f11att'], V['f11b'], 64))
    f = _elu(jnp.concatenate(
        [_gat(f, mTA, V['r12Wl'], V['r12Wr'], V['r12att'], V['r12b'], 64), f],
        axis=-1))
    f = _elu(_gat(f, mTA, V['f2Wl'], V['f2Wr'], V['f2att'], V['f2b'], 128))
    f = jnp.tanh(_dot(f, V['f3W']) + V['f3b'])
    fc_ref[...] = (jnp.sum(_rf(f) * _rf(V['f4Wr']), axis=1, keepdims=True)
                   + V['f4b'][0, 0])

    row = jax.lax.broadcasted_iota(jnp.int32, (NPAD, 1), 0)
    vmask = (row < NV).astype(jnp.float32)
    xpn = _dot(xcB, V['ndW'])
    Ep = _dot(EA, V['ndE']) * vmask
    epw0 = _dot(Ep, V['W0t'])
    hf = epw0 + _dot(xpn, V['W1t']) + V['a1b']
    h2 = _dot(jnp.tanh(hf), V['a2W']) + V['a2b']
    PM = _dot(h2, V['normP'])
    q2 = _dot(jnp.tanh(epw0 + V['a1b']), V['a2W']) + V['a2b']
    Q = _dot(q2, V['maskP'])

    lane = jax.lax.broadcasted_iota(jnp.int32, (1, NPAD), 1)

    def node_body(i, acc):
        rm = row == i
        qrow = jnp.sum(jnp.where(rm, Q, 0.0), axis=0, keepdims=True)
        pm = jnp.where(rm, qrow, PM)
        pm = _elu(_gat(pm, mTB, V['g1Wl'], V['g1Wr'], V['g1att'],
                       V['g1b'], 32))
        pm = _elu(_gat(pm, mTB, V['g2Wl'], V['g2Wr'], V['g2att'],
                       V['g2b'], 32))
        prow = jnp.sum(jnp.where(rm, pm, 0.0), axis=0, keepdims=True)
        val = jnp.sum(_rf(prow) * _rf(V['rWr'])) + V['rb'][0, 0]
        return jnp.where(lane == i, jnp.tanh(val), acc)

    nr_ref[...] = jax.lax.fori_loop(0, NV, node_body,
                                    jnp.zeros((1, NPAD), jnp.float32))


def kernel(x, edge, sensor_indx, params):
    P = params

    def pad_rows(a):
        return jnp.pad(a, ((0, NPAD - a.shape[0]),) + ((0, 0),) * (a.ndim - 1))

    def pad2(a):
        return jnp.pad(a, ((0, NPAD - a.shape[0]), (0, NPAD - a.shape[1])))

    ops = {
        'xA': pad_rows(x[-2]),
        'xB': pad_rows(x[-1]),
        'eTA': pad2(edge[-2]).T,
        'eTB': pad2(edge[-1]).T,
        'emb': pad_rows(P['emb'][sensor_indx]),
        'proj': pad_rows(P['proj'][0]),
        'r3W': P['r3W'], 'r3b': P['r3b'][None, :],
        'r4Wr': P['r4W'].T, 'r4b': P['r4b'][None, :],
        'f3W': P['f3W'], 'f3b': P['f3b'][None, :],
        'f4Wr': P['f4W'].T, 'f4b': P['f4b'][None, :],
        'ndW': P['nd_node_proj'], 'ndE': P['nd_emb_proj'],
        'W0t': P['nd_a1W'][:, :, 0].T, 'W1t': P['nd_a1W'][:, :, 1].T,
        'a1b': P['nd_a1b'][None, :],
        'a2W': P['nd_a2W'], 'a2b': P['nd_a2b'][None, :],
        'maskP': P['nd_mask_proj'], 'normP': P['nd_norm_proj'],
        'rWr': P['nd_rW'].T, 'rb': P['nd_rb'][None, :],
    }
    for nm in ('c1', 'c2', 'c3', 'c4', 'r11', 'r12', 'r2', 'f11', 'f2',
               'g1', 'g2'):
        g = P[nm]
        ops[nm + 'Wl'] = g['Wl']
        ops[nm + 'Wr'] = g['Wr']
        ops[nm + 'att'] = g['att']
        ops[nm + 'b'] = g['b'][None, :]

    names = list(ops.keys())
    vals = [ops[nm] for nm in names]

    recon, fc, nr = pl.pallas_call(
        lambda *refs: _body(names, *refs),
        out_shape=[
            jax.ShapeDtypeStruct((NPAD, 1), jnp.float32),
            jax.ShapeDtypeStruct((NPAD, 1), jnp.float32),
            jax.ShapeDtypeStruct((1, NPAD), jnp.float32),
        ],
    )(*vals)

    return (recon[:NV], fc[:NV], nr[0, :NV][:, None])


# speedup vs baseline: 1.2348x; 1.2018x over previous
import jax
import jax.numpy as jnp
from jax.experimental import pallas as pl

NPAD = 64
NV = 51
H = 4


def _leaky(z):
    return jnp.where(z >= 0, z, 0.2 * z)


def _elu(z):
    return jnp.where(z > 0, z, jnp.exp(jnp.where(z > 0, 0.0, z)) - 1.0)


def _b16(v):
    return v.astype(jnp.bfloat16)


def _rf(v):
    return v.astype(jnp.bfloat16).astype(jnp.float32)


def _dot(a, b):
    return jnp.dot(_b16(a), _b16(b), preferred_element_type=jnp.float32)


def _score3(xl, xr, attBD, d):
    z = xl[None, :, :] + xr[:, None, :]
    e = _rf(_leaky(z))
    s = _dot(e.reshape(NPAD * NPAD, H * d), attBD)
    return s.reshape(NPAD, NPAD, H)


def _softmax3(s3, mT3):
    s = jnp.where(mT3, s3, -1e9)
    s = s - jnp.max(s, axis=1, keepdims=True)
    ex = jnp.exp(s)
    a = ex / jnp.sum(ex, axis=1, keepdims=True)
    return jnp.where(mT3, a, 0.0)


def _aggregate(a3, xl, b, d):
    acc = _dot(a3[:, :, 0], xl[:, 0:d])
    for h in range(1, H):
        acc = acc + _dot(a3[:, :, h], xl[:, h * d:(h + 1) * d])
    return acc * (1.0 / H) + b


def _gat(xin, mT3, wl, wr, attBD, b, d):
    xl = _dot(xin, wl)
    xr = _dot(xin, wr)
    a3 = _softmax3(_score3(xl, xr, attBD, d), mT3)
    return _aggregate(a3, xl, b, d)


def _body(names, *refs):
    n_in = len(names)
    V = {nm: refs[i][...] for i, nm in enumerate(names)}
    recon_ref, fc_ref, nr_ref = refs[n_in:]

    mTA3 = V['eTA'][:, :, None] != 0.0
    mTB3 = V['eTB'][:, :, None] != 0.0
    mTBf = jnp.where(V['eTB'] != 0.0, 1.0, 0.0)

    def make_xc(xt):
        xp = jnp.sum(_rf(xt)[:, :, None] * _rf(V['proj']), axis=1)
        return jnp.concatenate([xp, V['emb']], axis=-1)

    xcA = make_xc(V['xA'])
    xcB = make_xc(V['xB'])

    def conv_stack(xc, mT3):
        o = _elu(_gat(xc, mT3, V['c1Wl'], V['c1Wr'], V['c1att'], V['c1b'],
                      128))
        E = _elu(_gat(o, mT3, V['c2Wl'], V['c2Wr'], V['c2att'], V['c2b'], 64))
        E = _elu(jnp.concatenate(
            [_gat(E, mT3, V['c3Wl'], V['c3Wr'], V['c3att'], V['c3b'], 64), E],
            axis=-1))
        E = _elu(_gat(E, mT3, V['c4Wl'], V['c4Wr'], V['c4att'], V['c4b'], 64))
        return E

    EA = conv_stack(xcA, mTA3)
    EB = conv_stack(xcB, mTB3)

    r = _elu(_gat(EB, mTB3, V['r11Wl'], V['r11Wr'], V['r11att'], V['r11b'],
                  64))
    r = _elu(jnp.concatenate(
        [_gat(r, mTB3, V['r12Wl'], V['r12Wr'], V['r12att'], V['r12b'], 64),
         r], axis=-1))
    r = _elu(_gat(r, mTB3, V['r2Wl'], V['r2Wr'], V['r2att'], V['r2b'], 128))
    r = jnp.tanh(_dot(r, V['r3W']) + V['r3b'])
    recon_ref[...] = (jnp.sum(_rf(r) * _rf(V['r4Wr']), axis=1, keepdims=True)
                      + V['r4b'][0, 0])

    f = _elu(_gat(EA, mTA3, V['f11Wl'], V['f11Wr'], V['f11att'], V['f11b'],
                  64))
    f = _elu(jnp.concatenate(
        [_gat(f, mTA3, V['r12Wl'], V['r12Wr'], V['r12att'], V['r12b'], 64),
         f], axis=-1))
    f = _elu(_gat(f, mTA3, V['f2Wl'], V['f2Wr'], V['f2att'], V['f2b'], 128))
    f = jnp.tanh(_dot(f, V['f3W']) + V['f3b'])
    fc_ref[...] = (jnp.sum(_rf(f) * _rf(V['f4Wr']), axis=1, keepdims=True)
                   + V['f4b'][0, 0])

    row = jax.lax.broadcasted_iota(jnp.int32, (NPAD, 1), 0)
    lane = jax.lax.broadcasted_iota(jnp.int32, (1, NPAD), 1)
    vmask = (row < NV).astype(jnp.float32)
    xpn = _dot(xcB, V['ndW'])
    Ep = _dot(EA, V['ndE']) * vmask
    epw0 = _dot(Ep, V['W0t'])
    hf = epw0 + _dot(xpn, V['W1t']) + V['a1b']
    h2 = _dot(jnp.tanh(hf), V['a2W']) + V['a2b']
    PM = _dot(h2, V['normP'])
    q2 = _dot(jnp.tanh(epw0 + V['a1b']), V['a2W']) + V['a2b']
    Q = _dot(q2, V['maskP'])

    XLf = _dot(PM, V['g1Wl'])
    XRf = _dot(PM, V['g1Wr'])
    XLq = _dot(Q, V['g1Wl'])
    XRq = _dot(Q, V['g1Wr'])
    Sf = _score3(XLf, XRf, V['g1att'], 32)
    Sc = _score3(XLq, XRf, V['g1att'], 32)
    Sr = _score3(XLf, XRq, V['g1att'], 32)
    zd = _rf(_leaky(XLq + XRq))
    S4 = _dot(zd, V['g1att'])

    Lf = jnp.where(mTB3, Sf, -1e9)
    tio = jax.lax.broadcasted_iota(jnp.int32, (NPAD, 1, 1), 0)
    sio = jax.lax.broadcasted_iota(jnp.int32, (1, NPAD, 1), 1)

    def node_body(i, acc):
        rm = row == i
        mcol = jnp.sum(jnp.where(lane == i, mTBf, 0.0), axis=1,
                       keepdims=True)
        vcol = jnp.sum(jnp.where(sio == i, Sc, 0.0), axis=1,
                       keepdims=True)
        vcol = jnp.where(mcol[:, :, None] > 0, vcol, -1e9)
        mrow = jnp.sum(jnp.where(rm, mTBf, 0.0), axis=0,
                       keepdims=True)
        vrow = jnp.sum(jnp.where(tio == i, Sr, 0.0), axis=0,
                       keepdims=True)
        vrow = jnp.where(mrow[:, :, None] > 0, vrow, -1e9)
        mc = jnp.sum(jnp.where(rm & (lane == i), mTBf, 0.0))
        vc = jnp.sum(jnp.where(rm, S4, 0.0), axis=0,
                     keepdims=True)[:, None, :]
        vc = jnp.where(mc > 0, vc, -1e9)

        L = jnp.where(sio == i, vcol, Lf)
        L = jnp.where(tio == i, vrow, L)
        L = jnp.where((tio == i) & (sio == i), vc, L)

        L = L - jnp.max(L, axis=1, keepdims=True)
        ex = jnp.exp(L)
        a3 = ex / jnp.sum(ex, axis=1, keepdims=True)
        a3 = jnp.where(mTB3, a3, 0.0)

        XLi = jnp.where(rm, XLq, XLf)
        pm = _elu(_aggregate(a3, XLi, V['g1b'], 32))
        pm = _elu(_gat(pm, mTB3, V['g2Wl'], V['g2Wr'], V['g2att'],
                       V['g2b'], 32))
        prow = jnp.sum(jnp.where(rm, pm, 0.0), axis=0, keepdims=True)
        val = jnp.sum(_rf(prow) * _rf(V['rWr'])) + V['rb'][0, 0]
        return jnp.where(lane == i, jnp.tanh(val), acc)

    nr_ref[...] = jax.lax.fori_loop(0, NV, node_body,
                                    jnp.zeros((1, NPAD), jnp.float32))


def kernel(x, edge, sensor_indx, params):
    P = params

    def pad_rows(a):
        return jnp.pad(a, ((0, NPAD - a.shape[0]),) + ((0, 0),) * (a.ndim - 1))

    def pad2(a):
        return jnp.pad(a, ((0, NPAD - a.shape[0]), (0, NPAD - a.shape[1])))

    def att_bd(att):
        d = att.shape[1]
        hh = jnp.arange(H)[:, None, None]
        col = jnp.arange(H)[None, None, :]
        blk = jnp.where(hh == col, att[:, :, None], 0.0)
        return blk.reshape(H * d, H)

    ops = {
        'xA': pad_rows(x[-2]),
        'xB': pad_rows(x[-1]),
        'eTA': pad2(edge[-2]).T,
        'eTB': pad2(edge[-1]).T,
        'emb': pad_rows(P['emb'][sensor_indx]),
        'proj': pad_rows(P['proj'][0]),
        'r3W': P['r3W'], 'r3b': P['r3b'][None, :],
        'r4Wr': P['r4W'].T, 'r4b': P['r4b'][None, :],
        'f3W': P['f3W'], 'f3b': P['f3b'][None, :],
        'f4Wr': P['f4W'].T, 'f4b': P['f4b'][None, :],
        'ndW': P['nd_node_proj'], 'ndE': P['nd_emb_proj'],
        'W0t': P['nd_a1W'][:, :, 0].T, 'W1t': P['nd_a1W'][:, :, 1].T,
        'a1b': P['nd_a1b'][None, :],
        'a2W': P['nd_a2W'], 'a2b': P['nd_a2b'][None, :],
        'maskP': P['nd_mask_proj'], 'normP': P['nd_norm_proj'],
        'rWr': P['nd_rW'].T, 'rb': P['nd_rb'][None, :],
    }
    for nm in ('c1', 'c2', 'c3', 'c4', 'r11', 'r12', 'r2', 'f11', 'f2',
               'g1', 'g2'):
        g = P[nm]
        ops[nm + 'Wl'] = g['Wl']
        ops[nm + 'Wr'] = g['Wr']
        ops[nm + 'att'] = att_bd(g['att'])
        ops[nm + 'b'] = g['b'][None, :]

    names = list(ops.keys())
    vals = [ops[nm] for nm in names]

    recon, fc, nr = pl.pallas_call(
        lambda *refs: _body(names, *refs),
        out_shape=[
            jax.ShapeDtypeStruct((NPAD, 1), jnp.float32),
            jax.ShapeDtypeStruct((NPAD, 1), jnp.float32),
            jax.ShapeDtypeStruct((1, NPAD), jnp.float32),
        ],
    )(*vals)

    return (recon[:NV], fc[:NV], nr[0, :NV][:, None])


# speedup vs baseline: 1.8369x; 1.4876x over previous
import jax
import jax.numpy as jnp
from jax.experimental import pallas as pl

NPAD = 64
NV = 51
H = 4


def _leaky(z):
    return jnp.where(z >= 0, z, 0.2 * z)


def _elu(z):
    return jnp.where(z > 0, z, jnp.exp(jnp.where(z > 0, 0.0, z)) - 1.0)


def _b16(v):
    return v.astype(jnp.bfloat16)


def _rf(v):
    return v.astype(jnp.bfloat16).astype(jnp.float32)


def _dot(a, b):
    return jnp.dot(_b16(a), _b16(b), preferred_element_type=jnp.float32)


def _score3(xl, xr, attBD, d):
    z = xl[None, :, :] + xr[:, None, :]
    e = _b16(_leaky(z))
    s = jnp.dot(e.reshape(NPAD * NPAD, H * d), _b16(attBD),
                preferred_element_type=jnp.float32)
    return s.reshape(NPAD, NPAD, H)


def _softmax3(s3, mT3):
    s = jnp.where(mT3, s3, -1e9)
    s = s - jnp.max(s, axis=1, keepdims=True)
    ex = jnp.exp(s)
    a = ex / jnp.sum(ex, axis=1, keepdims=True)
    return jnp.where(mT3, a, 0.0)


def _aggregate(a3, xl, b, d):
    acc = _dot(a3[:, :, 0], xl[:, 0:d])
    for h in range(1, H):
        acc = acc + _dot(a3[:, :, h], xl[:, h * d:(h + 1) * d])
    return acc * (1.0 / H) + b


def _gat(xin, mT3, wl, wr, attBD, b, d):
    xl = _dot(xin, wl)
    xr = _dot(xin, wr)
    a3 = _softmax3(_score3(xl, xr, attBD, d), mT3)
    return _aggregate(a3, xl, b, d)


def _body(names, *refs):
    n_in = len(names)
    V = {nm: refs[i][...] for i, nm in enumerate(names)}
    recon_ref, fc_ref, nr_ref = refs[n_in:]

    mTA3 = V['eTA'][:, :, None] != 0.0
    mTB3 = V['eTB'][:, :, None] != 0.0
    mTBf = jnp.where(V['eTB'] != 0.0, 1.0, 0.0)

    def make_xc(xt):
        xp = jnp.sum(_rf(xt)[:, :, None] * _rf(V['proj']), axis=1)
        return jnp.concatenate([xp, V['emb']], axis=-1)

    xcA = make_xc(V['xA'])
    xcB = make_xc(V['xB'])

    def conv_stack(xc, mT3):
        o = _elu(_gat(xc, mT3, V['c1Wl'], V['c1Wr'], V['c1att'], V['c1b'],
                      128))
        E = _elu(_gat(o, mT3, V['c2Wl'], V['c2Wr'], V['c2att'], V['c2b'], 64))
        E = _elu(jnp.concatenate(
            [_gat(E, mT3, V['c3Wl'], V['c3Wr'], V['c3att'], V['c3b'], 64), E],
            axis=-1))
        E = _elu(_gat(E, mT3, V['c4Wl'], V['c4Wr'], V['c4att'], V['c4b'], 64))
        return E

    EA = conv_stack(xcA, mTA3)
    EB = conv_stack(xcB, mTB3)

    r = _elu(_gat(EB, mTB3, V['r11Wl'], V['r11Wr'], V['r11att'], V['r11b'],
                  64))
    r = _elu(jnp.concatenate(
        [_gat(r, mTB3, V['r12Wl'], V['r12Wr'], V['r12att'], V['r12b'], 64),
         r], axis=-1))
    r = _elu(_gat(r, mTB3, V['r2Wl'], V['r2Wr'], V['r2att'], V['r2b'], 128))
    r = jnp.tanh(_dot(r, V['r3W']) + V['r3b'])
    recon_ref[...] = (jnp.sum(_rf(r) * _rf(V['r4Wr']), axis=1, keepdims=True)
                      + V['r4b'][0, 0])

    f = _elu(_gat(EA, mTA3, V['f11Wl'], V['f11Wr'], V['f11att'], V['f11b'],
                  64))
    f = _elu(jnp.concatenate(
        [_gat(f, mTA3, V['r12Wl'], V['r12Wr'], V['r12att'], V['r12b'], 64),
         f], axis=-1))
    f = _elu(_gat(f, mTA3, V['f2Wl'], V['f2Wr'], V['f2att'], V['f2b'], 128))
    f = jnp.tanh(_dot(f, V['f3W']) + V['f3b'])
    fc_ref[...] = (jnp.sum(_rf(f) * _rf(V['f4Wr']), axis=1, keepdims=True)
                   + V['f4b'][0, 0])

    row = jax.lax.broadcasted_iota(jnp.int32, (NPAD, 1), 0)
    lane = jax.lax.broadcasted_iota(jnp.int32, (1, NPAD), 1)
    vmask = (row < NV).astype(jnp.float32)
    xpn = _dot(xcB, V['ndW'])
    Ep = _dot(EA, V['ndE']) * vmask
    epw0 = _dot(Ep, V['W0t'])
    hf = epw0 + _dot(xpn, V['W1t']) + V['a1b']
    h2 = _dot(jnp.tanh(hf), V['a2W']) + V['a2b']
    PM = _dot(h2, V['normP'])
    q2 = _dot(jnp.tanh(epw0 + V['a1b']), V['a2W']) + V['a2b']
    Q = _dot(q2, V['maskP'])

    XLf = _dot(PM, V['g1Wl'])
    XRf = _dot(PM, V['g1Wr'])
    XLq = _dot(Q, V['g1Wl'])
    XRq = _dot(Q, V['g1Wr'])
    Sf = _score3(XLf, XRf, V['g1att'], 32)
    Sc = _score3(XLq, XRf, V['g1att'], 32)
    Sr = _score3(XLf, XRq, V['g1att'], 32)
    zd = _b16(_leaky(XLq + XRq))
    S4 = jnp.dot(zd, _b16(V['g1att']),
                 preferred_element_type=jnp.float32)
    eBm = jnp.where(V['eB'] != 0.0, 1.0, 0.0)

    Lf = jnp.where(mTB3, Sf, -1e9)
    tio = jax.lax.broadcasted_iota(jnp.int32, (NPAD, 1, 1), 0)
    sio = jax.lax.broadcasted_iota(jnp.int32, (1, NPAD, 1), 1)

    def node_body(i, acc):
        rm = row == i
        mcol = jnp.sum(jnp.where(lane == i, mTBf, 0.0), axis=1,
                       keepdims=True)
        vcol = jnp.sum(jnp.where(sio == i, Sc, 0.0), axis=1,
                       keepdims=True)
        vcol = jnp.where(mcol[:, :, None] > 0, vcol, -1e9)
        mrow = jnp.sum(jnp.where(rm, mTBf, 0.0), axis=0,
                       keepdims=True)
        vrow = jnp.sum(jnp.where(tio == i, Sr, 0.0), axis=0,
                       keepdims=True)
        vrow = jnp.where(mrow[:, :, None] > 0, vrow, -1e9)
        mc = jnp.sum(jnp.where(rm & (lane == i), mTBf, 0.0))
        vc = jnp.sum(jnp.where(rm, S4, 0.0), axis=0,
                     keepdims=True)[:, None, :]
        vc = jnp.where(mc > 0, vc, -1e9)

        L = jnp.where(sio == i, vcol, Lf)
        L = jnp.where(tio == i, vrow, L)
        L = jnp.where((tio == i) & (sio == i), vc, L)

        L = L - jnp.max(L, axis=1, keepdims=True)
        ex = jnp.exp(L)
        a3 = ex / jnp.sum(ex, axis=1, keepdims=True)
        a3 = jnp.where(mTB3, a3, 0.0)

        XLi = jnp.where(rm, XLq, XLf)
        pm = _elu(_aggregate(a3, XLi, V['g1b'], 32))

        xl2 = _dot(pm, V['g2Wl'])
        xr2 = _dot(pm, V['g2Wr'])
        xr2i = jnp.sum(jnp.where(rm, xr2, 0.0), axis=0, keepdims=True)
        ze = _b16(_leaky(xl2 + xr2i))
        srow = jnp.dot(ze, _b16(V['g2att']),
                       preferred_element_type=jnp.float32)
        mBcol = jnp.sum(jnp.where(lane == i, eBm, 0.0), axis=1,
                        keepdims=True)
        srow = jnp.where(mBcol > 0, srow, -1e9)
        srow = srow - jnp.max(srow, axis=0, keepdims=True)
        exr = jnp.exp(srow)
        a2 = exr / jnp.sum(exr, axis=0, keepdims=True)
        a2 = jnp.where(mBcol > 0, a2, 0.0)
        a2t = jnp.transpose(a2)
        o = jnp.zeros((1, 32), jnp.float32)
        for h in range(H):
            o = o + _dot(a2t[h:h + 1, :], xl2[:, h * 32:(h + 1) * 32])
        pm2row = _elu(o * (1.0 / H) + V['g2b'])
        val = jnp.sum(_rf(pm2row) * _rf(V['rWr'])) + V['rb'][0, 0]
        return jnp.where(lane == i, jnp.tanh(val), acc)

    nr_ref[...] = jax.lax.fori_loop(0, NV, node_body,
                                    jnp.zeros((1, NPAD), jnp.float32))


def kernel(x, edge, sensor_indx, params):
    P = params

    def pad_rows(a):
        return jnp.pad(a, ((0, NPAD - a.shape[0]),) + ((0, 0),) * (a.ndim - 1))

    def pad2(a):
        return jnp.pad(a, ((0, NPAD - a.shape[0]), (0, NPAD - a.shape[1])))

    def att_bd(att):
        d = att.shape[1]
        hh = jnp.arange(H)[:, None, None]
        col = jnp.arange(H)[None, None, :]
        blk = jnp.where(hh == col, att[:, :, None], 0.0)
        return blk.reshape(H * d, H)

    ops = {
        'xA': pad_rows(x[-2]),
        'xB': pad_rows(x[-1]),
        'eTA': pad2(edge[-2]).T,
        'eTB': pad2(edge[-1]).T,
        'eB': pad2(edge[-1]),
        'emb': pad_rows(P['emb'][sensor_indx]),
        'proj': pad_rows(P['proj'][0]),
        'r3W': P['r3W'], 'r3b': P['r3b'][None, :],
        'r4Wr': P['r4W'].T, 'r4b': P['r4b'][None, :],
        'f3W': P['f3W'], 'f3b': P['f3b'][None, :],
        'f4Wr': P['f4W'].T, 'f4b': P['f4b'][None, :],
        'ndW': P['nd_node_proj'], 'ndE': P['nd_emb_proj'],
        'W0t': P['nd_a1W'][:, :, 0].T, 'W1t': P['nd_a1W'][:, :, 1].T,
        'a1b': P['nd_a1b'][None, :],
        'a2W': P['nd_a2W'], 'a2b': P['nd_a2b'][None, :],
        'maskP': P['nd_mask_proj'], 'normP': P['nd_norm_proj'],
        'rWr': P['nd_rW'].T, 'rb': P['nd_rb'][None, :],
    }
    for nm in ('c1', 'c2', 'c3', 'c4', 'r11', 'r12', 'r2', 'f11', 'f2',
               'g1', 'g2'):
        g = P[nm]
        ops[nm + 'Wl'] = g['Wl']
        ops[nm + 'Wr'] = g['Wr']
        ops[nm + 'att'] = att_bd(g['att'])
        ops[nm + 'b'] = g['b'][None, :]

    names = list(ops.keys())
    vals = [ops[nm] for nm in names]

    recon, fc, nr = pl.pallas_call(
        lambda *refs: _body(names, *refs),
        out_shape=[
            jax.ShapeDtypeStruct((NPAD, 1), jnp.float32),
            jax.ShapeDtypeStruct((NPAD, 1), jnp.float32),
            jax.ShapeDtypeStruct((1, NPAD), jnp.float32),
        ],
    )(*vals)

    return (recon[:NV], fc[:NV], nr[0, :NV][:, None])
